# Initial kernel scaffold; baseline (speedup 1.0000x reference)
import jax
import jax.numpy as jnp
from jax.experimental import pallas as pl


def kernel(news_freshness, news_user_topic_lifetime, freshness_table, lifetime_table, W, b):
    raise NotImplementedError("write your pallas kernel here")


# speedup vs baseline: 3.5953x; 3.5953x over previous
import functools

import jax
import jax.numpy as jnp
import numpy as np
from jax.experimental import pallas as pl
from jax.experimental.pallas import tpu as pltpu

_NUM_BUCKETS = 100
_EMBED = 64
_HIDDEN = 128
_ROWS_PER_STEP = 2048


def _tc_body(nf_ref, nl_ref, ftp_ref, ltp_ref, wf_ref, wl_ref, b_ref,
             out_ref, pf_ref, pl_ref):
    @pl.when(pl.program_id(0) == 0)
    def _():
        pf_ref[...] = jnp.dot(ftp_ref[...], wf_ref[...],
                              preferred_element_type=jnp.float32) + b_ref[...]
        pl_ref[...] = jnp.dot(ltp_ref[...], wl_ref[...],
                              preferred_element_type=jnp.float32)

    ln_day = jnp.log(jnp.float32(60 * 60 * 24.0))
    scale = jnp.float32(_NUM_BUCKETS / 7)

    def bucketize(x):
        xf = jnp.clip(x.astype(jnp.float32), 1.0, None)
        scaled = jnp.log(xf) / ln_day
        bkt = (scaled * scale).astype(jnp.int32)
        return jnp.clip(bkt, None, _NUM_BUCKETS - 1)

    fb = bucketize(nf_ref[...])
    lb = bucketize(nl_ref[...])
    lanes = jax.lax.broadcasted_iota(jnp.int32, (1, _HIDDEN), 1)
    oh_f = (fb == lanes).astype(jnp.float32)
    oh_l = (lb == lanes).astype(jnp.float32)
    acc = jnp.dot(oh_f, pf_ref[...], preferred_element_type=jnp.float32)
    acc += jnp.dot(oh_l, pl_ref[...], preferred_element_type=jnp.float32)
    out_ref[...] = jnp.tanh(acc)


@jax.jit
def kernel(news_freshness, news_user_topic_lifetime, freshness_table,
           lifetime_table, W, b):
    batch, news = news_freshness.shape
    n = batch * news
    steps = n // _ROWS_PER_STEP

    nf = news_freshness.reshape(n, 1)
    nl = news_user_topic_lifetime.reshape(n, 1)
    pad = jnp.zeros((_HIDDEN - _NUM_BUCKETS, _EMBED), jnp.float32)
    ftp = jnp.concatenate([freshness_table, pad], axis=0)
    ltp = jnp.concatenate([lifetime_table, pad], axis=0)
    wf = W[:, :_EMBED].T
    wl = W[:, _EMBED:].T
    b2 = b.reshape(1, _HIDDEN)

    grid = (steps,)
    out = pl.pallas_call(
        _tc_body,
        grid=grid,
        in_specs=[
            pl.BlockSpec((_ROWS_PER_STEP, 1), lambda i: (i, 0)),
            pl.BlockSpec((_ROWS_PER_STEP, 1), lambda i: (i, 0)),
            pl.BlockSpec((_HIDDEN, _EMBED), lambda i: (0, 0)),
            pl.BlockSpec((_HIDDEN, _EMBED), lambda i: (0, 0)),
            pl.BlockSpec((_EMBED, _HIDDEN), lambda i: (0, 0)),
            pl.BlockSpec((_EMBED, _HIDDEN), lambda i: (0, 0)),
            pl.BlockSpec((1, _HIDDEN), lambda i: (0, 0)),
        ],
        out_specs=pl.BlockSpec((_ROWS_PER_STEP, _HIDDEN), lambda i: (i, 0)),
        out_shape=jax.ShapeDtypeStruct((n, _HIDDEN), jnp.float32),
        scratch_shapes=[
            pltpu.VMEM((_HIDDEN, _HIDDEN), jnp.float32),
            pltpu.VMEM((_HIDDEN, _HIDDEN), jnp.float32),
        ],
        compiler_params=pltpu.CompilerParams(
            dimension_semantics=("arbitrary",),
        ),
    )(nf, nl, ftp, ltp, wf, wl, b2)
    return out.reshape(batch, news, _HIDDEN)


# speedup vs baseline: 5.3303x; 1.4826x over previous
import functools

import jax
import jax.numpy as jnp
import numpy as np
from jax.experimental import pallas as pl
from jax.experimental.pallas import tpu as pltpu

_NUM_BUCKETS = 100
_EMBED = 64
_HIDDEN = 128
_ROWS_PER_STEP = 2048


def _tc_body(nf_ref, nl_ref, ftp_ref, ltp_ref, wf_ref, wl_ref, b_ref,
             out_ref, pf_ref, pl_ref):
    @pl.when(pl.program_id(0) == 0)
    def _():
        pf_ref[...] = jnp.dot(ftp_ref[...], wf_ref[...],
                              preferred_element_type=jnp.float32) + b_ref[...]
        pl_ref[...] = jnp.dot(ltp_ref[...], wl_ref[...],
                              preferred_element_type=jnp.float32)

    ln_day = jnp.log(jnp.float32(60 * 60 * 24.0))
    scale = jnp.float32(_NUM_BUCKETS / 7)

    def bucketize(x):
        xf = jnp.clip(x.astype(jnp.float32), 1.0, None)
        scaled = jnp.log(xf) / ln_day
        bkt = (scaled * scale).astype(jnp.int32)
        return jnp.clip(bkt, None, _NUM_BUCKETS - 1)

    fb = bucketize(nf_ref[0])
    lb = bucketize(nl_ref[0])
    rows = jax.lax.broadcasted_iota(jnp.int32, (_HIDDEN, 1), 0)
    oh_f = (fb == rows).astype(jnp.float32)
    oh_l = (lb == rows).astype(jnp.float32)
    dn = (((0,), (0,)), ((), ()))
    acc = jax.lax.dot_general(oh_f, pf_ref[...], dn,
                              preferred_element_type=jnp.float32)
    acc += jax.lax.dot_general(oh_l, pl_ref[...], dn,
                               preferred_element_type=jnp.float32)
    out_ref[...] = jnp.tanh(acc)


@jax.jit
def kernel(news_freshness, news_user_topic_lifetime, freshness_table,
           lifetime_table, W, b):
    batch, news = news_freshness.shape
    n = batch * news
    steps = n // _ROWS_PER_STEP

    nf = news_freshness.reshape(steps, 1, _ROWS_PER_STEP)
    nl = news_user_topic_lifetime.reshape(steps, 1, _ROWS_PER_STEP)
    pad = jnp.zeros((_HIDDEN - _NUM_BUCKETS, _EMBED), jnp.float32)
    ftp = jnp.concatenate([freshness_table, pad], axis=0)
    ltp = jnp.concatenate([lifetime_table, pad], axis=0)
    wf = W[:, :_EMBED].T
    wl = W[:, _EMBED:].T
    b2 = b.reshape(1, _HIDDEN)

    grid = (steps,)
    out = pl.pallas_call(
        _tc_body,
        grid=grid,
        in_specs=[
            pl.BlockSpec((1, 1, _ROWS_PER_STEP), lambda i: (i, 0, 0)),
            pl.BlockSpec((1, 1, _ROWS_PER_STEP), lambda i: (i, 0, 0)),
            pl.BlockSpec((_HIDDEN, _EMBED), lambda i: (0, 0)),
            pl.BlockSpec((_HIDDEN, _EMBED), lambda i: (0, 0)),
            pl.BlockSpec((_EMBED, _HIDDEN), lambda i: (0, 0)),
            pl.BlockSpec((_EMBED, _HIDDEN), lambda i: (0, 0)),
            pl.BlockSpec((1, _HIDDEN), lambda i: (0, 0)),
        ],
        out_specs=pl.BlockSpec((_ROWS_PER_STEP, _HIDDEN), lambda i: (i, 0)),
        out_shape=jax.ShapeDtypeStruct((n, _HIDDEN), jnp.float32),
        scratch_shapes=[
            pltpu.VMEM((_HIDDEN, _HIDDEN), jnp.float32),
            pltpu.VMEM((_HIDDEN, _HIDDEN), jnp.float32),
        ],
        compiler_params=pltpu.CompilerParams(
            dimension_semantics=("arbitrary",),
        ),
    )(nf, nl, ftp, ltp, wf, wl, b2)
    return out.reshape(batch, news, _HIDDEN)
